# baseline (speedup 1.0000x reference)
import jax
import jax.numpy as jnp
from jax.experimental import pallas as pl
from jax.experimental.pallas import tpu as pltpu

B, K, N = 64, 16, 32768
SUB, LANE = 8, 4096
CH = 1024
NCH = LANE // CH
RB = 16
NB = 2

KEY0 = 928981903
KEY1 = 3453687069


def _threefry_keyed(x0, x1):
    k0 = jnp.uint32(KEY0)
    k1 = jnp.uint32(KEY1)
    k2 = jnp.uint32(KEY0 ^ KEY1 ^ 0x1BD11BDA)
    ks = (k0, k1, k2)
    rots = ((13, 15, 26, 6), (17, 29, 16, 24))

    def rotl(x, r):
        return (x << jnp.uint32(r)) | (x >> jnp.uint32(32 - r))

    for g in range(5):
        for r in rots[g % 2]:
            x0 = x0 + x1
            x1 = rotl(x1, r)
            x1 = x1 ^ x0
        x0 = x0 + ks[(g + 1) % 3]
        x1 = x1 + ks[(g + 2) % 3] + jnp.uint32(g + 1)
    return x0, x1


def _csamples_kernel(logits_ref, csamples_ref, e8_ref, q2a_ref, q2b_ref):
    g = pl.program_id(0)

    r_iota = jax.lax.broadcasted_iota(jnp.uint32, (SUB, CH), 0)
    c_iota = jax.lax.broadcasted_iota(jnp.uint32, (SUB, CH), 1)
    chunkflat = r_iota * jnp.uint32(LANE) + c_iota
    eps = jnp.finfo(jnp.float32).eps
    zero = jnp.zeros((SUB, CH), jnp.float32)

    for bl in range(NB):
        macc = jnp.full((SUB, CH), -jnp.inf, jnp.float32)
        for c in range(NCH):
            sl = pl.ds(c * CH, CH)
            macc = jnp.maximum(macc, logits_ref[bl, :, sl])
        lmax = jnp.max(macc)
        for c in range(NCH):
            sl = pl.ds(c * CH, CH)
            e8_ref[:, sl] = jnp.exp(logits_ref[bl, :, sl] - lmax)
            csamples_ref[bl, :, sl] = zero
            q2b_ref[:, sl] = zero

        b = g * jnp.int32(NB) + jnp.int32(bl)
        row0 = jax.lax.convert_element_type(b, jnp.uint32) * jnp.uint32(K)

        def gen_row(ku, dst_ref, prev_ref, rs_prev):
            base = (row0 + ku) * jnp.uint32(N) + jnp.uint32(KEY1)
            ssv = jnp.zeros((SUB, CH), jnp.float32)
            for c in range(NCH):
                sl = pl.ds(c * CH, CH)
                x1 = chunkflat + (base + jnp.uint32(c * CH))
                o0, o1 = _threefry_keyed(
                    jnp.full((SUB, CH), KEY0, jnp.uint32), x1
                )
                bits = o0 ^ o1
                mant = (bits >> jnp.uint32(9)) | jnp.uint32(0x3F800000)
                u = jax.lax.bitcast_convert_type(mant, jnp.float32) - 1.0
                u = jnp.clip(u, eps, 1.0 - eps)
                w = -jnp.log(u)
                q = e8_ref[:, sl] / w
                q2 = q * q
                dst_ref[:, sl] = q2
                ssv = ssv + q2
                csamples_ref[bl, :, sl] = jnp.maximum(
                    csamples_ref[bl, :, sl], prev_ref[:, sl] * rs_prev
                )
            return 1.0 / jnp.sum(ssv)

        def k_body(i, rs_b):
            ka = jax.lax.convert_element_type(2 * i, jnp.uint32)
            rs_a = gen_row(ka, q2a_ref, q2b_ref, rs_b)
            rs_b = gen_row(ka + jnp.uint32(1), q2b_ref, q2a_ref, rs_a)
            return rs_b

        rs_last = jax.lax.fori_loop(0, K // 2, k_body, jnp.float32(0.0))
        for c in range(NCH):
            sl = pl.ds(c * CH, CH)
            csamples_ref[bl, :, sl] = jnp.maximum(
                csamples_ref[bl, :, sl], q2b_ref[:, sl] * rs_last
            )


def _dsamples_kernel(logits_ref, dsamples_ref):
    l_rows = logits_ref[...]

    lb = jax.lax.bitcast_convert_type(l_rows + 0.0, jnp.uint32)
    neg = (lb >> jnp.uint32(31)) == jnp.uint32(1)
    ukey = jnp.where(neg, ~lb, lb | jnp.uint32(0x80000000))
    t = jnp.zeros((RB, 1), jnp.uint32)
    for bit in range(31, -1, -1):
        cand = t | jnp.uint32(1 << bit)
        cnt = jnp.sum((ukey >= cand).astype(jnp.int32), axis=1, keepdims=True)
        t = jnp.where(cnt >= K, cand, t)
    dsamples_ref[...] = (ukey >= t).astype(jnp.float32)


def kernel(logits):
    l_tiles = logits.reshape(B, SUB, LANE)
    spec_tile = pl.BlockSpec((NB, SUB, LANE), lambda b: (b, 0, 0))
    csamples = pl.pallas_call(
        _csamples_kernel,
        grid=(B // NB,),
        in_specs=[spec_tile],
        out_specs=spec_tile,
        out_shape=jax.ShapeDtypeStruct((B, SUB, LANE), jnp.float32),
        scratch_shapes=[
            pltpu.VMEM((SUB, LANE), jnp.float32),
            pltpu.VMEM((SUB, LANE), jnp.float32),
            pltpu.VMEM((SUB, LANE), jnp.float32),
        ],
        compiler_params=pltpu.CompilerParams(
            dimension_semantics=("parallel",),
        ),
    )(l_tiles)

    l2d = logits.reshape(B, N)
    spec_blk = pl.BlockSpec((RB, N), lambda b: (b, 0))
    dsamples = pl.pallas_call(
        _dsamples_kernel,
        grid=(B // RB,),
        in_specs=[spec_blk],
        out_specs=spec_blk,
        out_shape=jax.ShapeDtypeStruct((B, N), jnp.float32),
        compiler_params=pltpu.CompilerParams(
            dimension_semantics=("parallel",),
        ),
    )(l2d)
    return dsamples, csamples.reshape(B, N)


# speedup vs baseline: 1.0041x; 1.0041x over previous
import jax
import jax.numpy as jnp
from jax.experimental import pallas as pl
from jax.experimental.pallas import tpu as pltpu

B, K, N = 64, 16, 32768
SUB, LANE = 8, 4096
CH = 1024
NCH = LANE // CH
RB = 16

KEY0 = 928981903
KEY1 = 3453687069


def _threefry_keyed(x0, x1):
    k0 = jnp.uint32(KEY0)
    k1 = jnp.uint32(KEY1)
    k2 = jnp.uint32(KEY0 ^ KEY1 ^ 0x1BD11BDA)
    ks = (k0, k1, k2)
    rots = ((13, 15, 26, 6), (17, 29, 16, 24))

    def rotl(x, r):
        return (x << jnp.uint32(r)) | (x >> jnp.uint32(32 - r))

    for g in range(5):
        for r in rots[g % 2]:
            x0 = x0 + x1
            x1 = rotl(x1, r)
            x1 = x1 ^ x0
        x0 = x0 + ks[(g + 1) % 3]
        x1 = x1 + ks[(g + 2) % 3] + jnp.uint32(g + 1)
    return x0, x1


def _csamples_kernel(logits_ref, csamples_ref, e8_ref, q2a_ref, q2b_ref):
    b = pl.program_id(0)

    macc = jnp.full((SUB, CH), -jnp.inf, jnp.float32)
    for c in range(NCH):
        sl = pl.ds(c * CH, CH)
        macc = jnp.maximum(macc, logits_ref[0, :, sl])
    lmax = jnp.max(macc)
    zero = jnp.zeros((SUB, CH), jnp.float32)
    for c in range(NCH):
        sl = pl.ds(c * CH, CH)
        e8_ref[:, sl] = jnp.exp(logits_ref[0, :, sl] - lmax)
        csamples_ref[0, :, sl] = zero
        q2b_ref[:, sl] = zero

    r_iota = jax.lax.broadcasted_iota(jnp.uint32, (SUB, CH), 0)
    c_iota = jax.lax.broadcasted_iota(jnp.uint32, (SUB, CH), 1)
    chunkflat = r_iota * jnp.uint32(LANE) + c_iota

    row0 = jax.lax.convert_element_type(b, jnp.uint32) * jnp.uint32(K)
    eps = jnp.finfo(jnp.float32).eps

    def gen_row(ku, dst_ref, prev_ref, rs_prev):
        base = (row0 + ku) * jnp.uint32(N) + jnp.uint32(KEY1)
        ssv = jnp.zeros((SUB, CH), jnp.float32)
        for c in range(NCH):
            sl = pl.ds(c * CH, CH)
            x1 = chunkflat + (base + jnp.uint32(c * CH))
            o0, o1 = _threefry_keyed(jnp.full((SUB, CH), KEY0, jnp.uint32), x1)
            bits = o0 ^ o1
            mant = (bits >> jnp.uint32(9)) | jnp.uint32(0x3F800000)
            u = jax.lax.bitcast_convert_type(mant, jnp.float32) - 1.0
            u = jnp.clip(u, eps, 1.0 - eps)
            w = -jnp.log(u)
            q = e8_ref[:, sl] / w
            q2 = q * q
            dst_ref[:, sl] = q2
            ssv = ssv + q2
            csamples_ref[0, :, sl] = jnp.maximum(
                csamples_ref[0, :, sl], prev_ref[:, sl] * rs_prev
            )
        return 1.0 / jnp.sum(ssv)

    def k_body(i, rs_b):
        ka = jax.lax.convert_element_type(2 * i, jnp.uint32)
        rs_a = gen_row(ka, q2a_ref, q2b_ref, rs_b)
        rs_b = gen_row(ka + jnp.uint32(1), q2b_ref, q2a_ref, rs_a)
        return rs_b

    rs_last = jax.lax.fori_loop(0, K // 2, k_body, jnp.float32(0.0))
    for c in range(NCH):
        sl = pl.ds(c * CH, CH)
        csamples_ref[0, :, sl] = jnp.maximum(
            csamples_ref[0, :, sl], q2b_ref[:, sl] * rs_last
        )


def _dsamples_kernel(logits_ref, dsamples_ref):
    l_rows = logits_ref[...]

    lb = jax.lax.bitcast_convert_type(l_rows + 0.0, jnp.uint32)
    neg = (lb >> jnp.uint32(31)) == jnp.uint32(1)
    ukey = jnp.where(neg, ~lb, lb | jnp.uint32(0x80000000))
    t = jnp.zeros((RB, 1), jnp.uint32)
    for bit in range(31, -1, -1):
        cand = t | jnp.uint32(1 << bit)
        cnt = jnp.sum((ukey >= cand).astype(jnp.int32), axis=1, keepdims=True)
        t = jnp.where(cnt >= K, cand, t)
    dsamples_ref[...] = (ukey >= t).astype(jnp.float32)


def kernel(logits):
    l_tiles = logits.reshape(B, SUB, LANE)
    spec_tile = pl.BlockSpec((1, SUB, LANE), lambda b: (b, 0, 0))
    csamples = pl.pallas_call(
        _csamples_kernel,
        grid=(B,),
        in_specs=[spec_tile],
        out_specs=spec_tile,
        out_shape=jax.ShapeDtypeStruct((B, SUB, LANE), jnp.float32),
        scratch_shapes=[
            pltpu.VMEM((SUB, LANE), jnp.float32),
            pltpu.VMEM((SUB, LANE), jnp.float32),
            pltpu.VMEM((SUB, LANE), jnp.float32),
        ],
        compiler_params=pltpu.CompilerParams(
            dimension_semantics=("parallel",),
        ),
    )(l_tiles)

    l2d = logits.reshape(B, N)
    spec_blk = pl.BlockSpec((RB, N), lambda b: (b, 0))
    dsamples = pl.pallas_call(
        _dsamples_kernel,
        grid=(B // RB,),
        in_specs=[spec_blk],
        out_specs=spec_blk,
        out_shape=jax.ShapeDtypeStruct((B, N), jnp.float32),
        compiler_params=pltpu.CompilerParams(
            dimension_semantics=("parallel",),
        ),
    )(l2d)
    return dsamples, csamples.reshape(B, N)
